# baseline (speedup 1.0000x reference)
import jax
import jax.numpy as jnp
from jax.experimental import pallas as pl
from jax.experimental.pallas import tpu as pltpu

_B, _N, _F, _D = 4, 100000, 200000, 128
_ROWS = _B * _N


def _body(x_any, face_any, face_vmem, verts_any, faces_any, edges_ref,
          sem_v, sem_f):
    i = pl.program_id(0)

    @pl.when(i == 0)
    def _():
        pltpu.make_async_copy(x_any, verts_any, sem_v).start()
        pltpu.make_async_copy(face_any, faces_any, sem_f).start()

    for r in range(2):
        for s in range(3):
            c = s if r == 0 else (s + 1) % 3
            edges_ref[0, r, pl.ds(s * _F, _F)] = face_vmem[c, i, :]

    @pl.when(i == _B - 1)
    def _():
        pltpu.make_async_copy(x_any, verts_any, sem_v).wait()
        pltpu.make_async_copy(face_any, faces_any, sem_f).wait()


def kernel(x, mask, face):
    del mask
    face_t = jnp.transpose(face, (2, 0, 1))
    x2 = x.reshape(_ROWS, _D)
    verts2, faces_t, new_edges = pl.pallas_call(
        _body,
        grid=(_B,),
        in_specs=[
            pl.BlockSpec(memory_space=pltpu.MemorySpace.HBM),
            pl.BlockSpec(memory_space=pltpu.MemorySpace.HBM),
            pl.BlockSpec((3, _B, _F), lambda b: (0, 0, 0)),
        ],
        out_specs=[
            pl.BlockSpec(memory_space=pltpu.MemorySpace.HBM),
            pl.BlockSpec(memory_space=pltpu.MemorySpace.HBM),
            pl.BlockSpec((1, 2, 3 * _F), lambda b: (b, 0, 0)),
        ],
        out_shape=(
            jax.ShapeDtypeStruct((_ROWS, _D), jnp.float32),
            jax.ShapeDtypeStruct((3, _B, _F), jnp.int32),
            jax.ShapeDtypeStruct((_B, 2, 3 * _F), jnp.int32),
        ),
        scratch_shapes=[pltpu.SemaphoreType.DMA, pltpu.SemaphoreType.DMA],
    )(x2, face_t, face_t)
    new_verts = verts2.reshape(_B, _N, _D)
    new_faces = jnp.transpose(faces_t, (1, 2, 0))
    return (new_verts, new_faces, new_edges)


# speedup vs baseline: 42.1684x; 42.1684x over previous
import jax
import jax.numpy as jnp
from jax.experimental import pallas as pl

_B, _N, _F, _D = 4, 100000, 200000, 128
_ROWS = _B * _N
_RBLK = 4000


def _body(face_vmem, x_blk, verts_blk, faces_blk, edges_blk):
    i = pl.program_id(0)
    verts_blk[...] = x_blk[...]

    @pl.when(i < 3)
    def _():
        faces_blk[...] = face_vmem[pl.ds(i, 1)]

    @pl.when(i < _B)
    def _():
        for r in range(2):
            for s in range(3):
                c = s if r == 0 else (s + 1) % 3
                edges_blk[0, r, pl.ds(s * _F, _F)] = face_vmem[c, i, :]


def kernel(x, mask, face):
    del mask
    face_t = jnp.transpose(face, (2, 0, 1))
    x2 = x.reshape(_ROWS, _D)
    verts2, faces_t, new_edges = pl.pallas_call(
        _body,
        grid=(_ROWS // _RBLK,),
        in_specs=[
            pl.BlockSpec((3, _B, _F), lambda i: (0, 0, 0)),
            pl.BlockSpec((_RBLK, _D), lambda i: (i, 0)),
        ],
        out_specs=[
            pl.BlockSpec((_RBLK, _D), lambda i: (i, 0)),
            pl.BlockSpec((1, _B, _F), lambda i: (jnp.minimum(i, 2), 0, 0)),
            pl.BlockSpec((1, 2, 3 * _F),
                         lambda i: (jnp.minimum(i, _B - 1), 0, 0)),
        ],
        out_shape=(
            jax.ShapeDtypeStruct((_ROWS, _D), jnp.float32),
            jax.ShapeDtypeStruct((3, _B, _F), jnp.int32),
            jax.ShapeDtypeStruct((_B, 2, 3 * _F), jnp.int32),
        ),
    )(face_t, x2)
    new_verts = verts2.reshape(_B, _N, _D)
    new_faces = jnp.transpose(faces_t, (1, 2, 0))
    return (new_verts, new_faces, new_edges)


# speedup vs baseline: 44.1630x; 1.0473x over previous
import jax
import jax.numpy as jnp
from jax.experimental import pallas as pl

_B, _N, _F, _D = 4, 100000, 200000, 128


def _edges_body(face_ref, out_ref):
    b = pl.program_id(0)
    for r in range(2):
        for s in range(3):
            c = s if r == 0 else (s + 1) % 3
            out_ref[0, r, pl.ds(s * _F, _F)] = face_ref[c, b, :]


def _edges_call(face_t):
    return pl.pallas_call(
        _edges_body,
        grid=(_B,),
        in_specs=[pl.BlockSpec((3, _B, _F), lambda b: (0, 0, 0))],
        out_specs=pl.BlockSpec((1, 2, 3 * _F), lambda b: (b, 0, 0)),
        out_shape=jax.ShapeDtypeStruct((_B, 2, 3 * _F), jnp.int32),
    )(face_t)


def _copy_body(src_ref, dst_ref):
    dst_ref[...] = src_ref[...]


def _faces_call(face_t):
    return pl.pallas_call(
        _copy_body,
        grid=(3,),
        in_specs=[pl.BlockSpec((1, _B, _F), lambda c: (c, 0, 0))],
        out_specs=pl.BlockSpec((1, _B, _F), lambda c: (c, 0, 0)),
        out_shape=jax.ShapeDtypeStruct((3, _B, _F), jnp.int32),
    )(face_t)


_ROWS = _B * _N
_RBLK = 16000


def _verts_copy(x2):
    return pl.pallas_call(
        _copy_body,
        grid=(_ROWS // _RBLK,),
        in_specs=[pl.BlockSpec((_RBLK, _D), lambda i: (i, 0))],
        out_specs=pl.BlockSpec((_RBLK, _D), lambda i: (i, 0)),
        out_shape=jax.ShapeDtypeStruct((_ROWS, _D), jnp.float32),
    )(x2)


def kernel(x, mask, face):
    del mask
    face_t = jnp.transpose(face, (2, 0, 1))
    new_edges = _edges_call(face_t)
    new_faces = jnp.transpose(_faces_call(face_t), (1, 2, 0))
    new_verts = _verts_copy(x.reshape(_ROWS, _D)).reshape(_B, _N, _D)
    return (new_verts, new_faces, new_edges)


# speedup vs baseline: 44.1876x; 1.0006x over previous
import jax
import jax.numpy as jnp
from jax.experimental import pallas as pl

_B, _N, _F, _D = 4, 100000, 200000, 128


def _edges_body(face_ref, out_ref):
    b = pl.program_id(0)
    for r in range(2):
        for s in range(3):
            c = s if r == 0 else (s + 1) % 3
            out_ref[0, r, pl.ds(s * _F, _F)] = face_ref[c, b, :]


def _edges_call(face_t):
    return pl.pallas_call(
        _edges_body,
        grid=(_B,),
        in_specs=[pl.BlockSpec((3, _B, _F), lambda b: (0, 0, 0))],
        out_specs=pl.BlockSpec((1, 2, 3 * _F), lambda b: (b, 0, 0)),
        out_shape=jax.ShapeDtypeStruct((_B, 2, 3 * _F), jnp.int32),
    )(face_t)


def _copy_body(src_ref, dst_ref):
    dst_ref[...] = src_ref[...]


def _faces_call(face_t):
    return pl.pallas_call(
        _copy_body,
        grid=(3,),
        in_specs=[pl.BlockSpec((1, _B, _F), lambda c: (c, 0, 0))],
        out_specs=pl.BlockSpec((1, _B, _F), lambda c: (c, 0, 0)),
        out_shape=jax.ShapeDtypeStruct((3, _B, _F), jnp.int32),
    )(face_t)


_ROWS = _B * _N
_RBLK = 20000


def _verts_copy(x2):
    return pl.pallas_call(
        _copy_body,
        grid=(_ROWS // _RBLK,),
        in_specs=[pl.BlockSpec((_RBLK, _D), lambda i: (i, 0))],
        out_specs=pl.BlockSpec((_RBLK, _D), lambda i: (i, 0)),
        out_shape=jax.ShapeDtypeStruct((_ROWS, _D), jnp.float32),
    )(x2)


def kernel(x, mask, face):
    del mask
    face_t = jnp.transpose(face, (2, 0, 1))
    new_edges = _edges_call(face_t)
    new_faces = jnp.transpose(_faces_call(face_t), (1, 2, 0))
    new_verts = _verts_copy(x.reshape(_ROWS, _D)).reshape(_B, _N, _D)
    return (new_verts, new_faces, new_edges)
